# baseline (speedup 1.0000x reference)
import jax
import jax.numpy as jnp
from jax import lax
from jax.experimental import pallas as pl
from jax.experimental.pallas import tpu as pltpu
from jax.experimental.pallas import tpu_sc as plsc

_N_USERS = 30000
_N_NODES = 50000
_E = 800000
_D = 64
_NSUB = 16
_EPAD = 819200
_ET = _EPAD // _NSUB
_HN = 25600
_CH = 64
_QCAP = 52224
_QTOT = 16 * _QCAP
_XR = 51200
_TR = _HN // _NSUB
_B = 4096


def _sc_body(x0, srcp, dstp, wp, usr, itm, zer, xall, gout,
             acc, r0, r1, r2, r3,
             sa, da, wa, sb, db, wb, dsc0, dsc1, uidx, iidx, gbuf,
             gsem, ssem, isem):
  c = lax.axis_index("c")
  s = lax.axis_index("s")
  rbufs = (r0, r1, r2, r3)
  lanev = lax.iota(jnp.int32, 16)
  gdims = lax.GatherDimensionNumbers(
      offset_dims=(), collapsed_slice_dims=(0,), start_index_map=(0,))

  def vperm(v, idx):
    return lax.gather(v, idx[:, None], gdims, (1,),
                      mode=lax.GatherScatterMode.PROMISE_IN_BOUNDS)

  rs = s * _TR
  qbase = s * _QCAP
  lo = c * _HN

  pltpu.sync_copy(zer, acc.at[pl.ds(rs, _TR)])
  plsc.subcore_barrier()

  nb = _ET // 1024

  def fire_group(eoff, sbuf, dbuf, wbuf_):
    pltpu.async_copy(srcp.at[pl.ds(s * _ET + eoff, 512)], sbuf, isem)
    pltpu.async_copy(dstp.at[pl.ds(s * _ET + eoff, 512)], dbuf, isem)
    pltpu.async_copy(wp.at[pl.ds(s * _ET + eoff, 512)], wbuf_, isem)

  def wait_group(sbuf, dbuf, wbuf_, loff):
    pltpu.make_async_copy(srcp.at[pl.ds(0, 512)], sbuf, isem).wait()
    pltpu.make_async_copy(dstp.at[pl.ds(0, 512)], dbuf, isem).wait()
    pltpu.make_async_copy(wp.at[pl.ds(0, 512)], wbuf_, isem).wait()

    def offr(k, cy):
      sbuf[pl.ds(k * 16, 16)] = sbuf[pl.ds(k * 16, 16)] + loff
      dv = dbuf[pl.ds(k * 16, 16)]
      dl = dv - lo
      kp = (dv >= lo) & (dl < _HN)
      dbuf[pl.ds(k * 16, 16)] = jnp.where(kp, dl, _HN + (dv & 2047))
      return cy

    lax.fori_loop(0, 32, offr, 0)

  def fire_g(l, sbuf, off, rbuf):
    @pl.when(l == 0)
    def _():
      pltpu.async_copy(x0.at[sbuf.at[pl.ds(off, _CH)]], rbuf, gsem)

    @pl.when(l > 0)
    def _():
      pltpu.async_copy(xall.at[sbuf.at[pl.ds(off, _CH)]], rbuf, gsem)

  def wait_g(rbuf):
    pltpu.make_async_copy(x0.at[pl.ds(0, _CH)], rbuf, gsem).wait()

  def scale_chunk(rbuf, wbuf_, woff):
    def scale(g, cy):
      wv = wbuf_[pl.ds(woff + g * 16, 16)]
      for k in range(16):
        e = g * 16 + k
        w = wv[k]
        for h in (0, 16, 32, 48):
          rbuf[e, pl.ds(h, 16)] = rbuf[e, pl.ds(h, 16)] * w
      return cy

    lax.fori_loop(0, _CH // 16, scale, 0)

  def fire_scatter(rbuf, dbuf, off, dscb):
    for k in range(4):
      dscb[pl.ds(k * 16, 16)] = dbuf[pl.ds(off + k * 16, 16)]
    pltpu.async_copy(rbuf, acc.at[dscb], ssem, add=True)

  def wait_scatter():
    pltpu.make_async_copy(r0, acc.at[pl.ds(0, _CH)], ssem).wait()

  def layer(l, lcarry):
    loff = jnp.where(l == 0, 0, (l - 1) * _XR)

    def body(b, carry):
      base = b * 1024
      for cc in range(16):
        rb = rbufs[cc % 4]
        sbuf, dbuf, wbuf_ = (sb, db, wb) if cc >= 8 else (sa, da, wa)
        off = (cc % 8) * _CH
        if cc == 2:
          fire_group(base + 512, sb, db, wb)
        wait_g(rb)
        scale_chunk(rb, wbuf_, off)
        if cc == 0:
          @pl.when(b > 0)
          def _():
            wait_scatter()
        else:
          wait_scatter()
        fire_scatter(rb, dbuf, off, dsc0 if cc % 2 == 0 else dsc1)
        if cc == 5:
          wait_group(sb, db, wb, loff)
        if cc == 10:
          @pl.when(b < nb - 1)
          def _():
            fire_group(base + 1024, sa, da, wa)
        if cc == 13:
          @pl.when(b < nb - 1)
          def _():
            wait_group(sa, da, wa, loff)
        nrb = rbufs[(cc + 2) % 4]
        if cc < 6:
          fire_g(l, sa, (cc + 2) * _CH, nrb)
        elif cc < 14:
          fire_g(l, sb, (cc - 6) * _CH, nrb)
        else:
          @pl.when(b < nb - 1)
          def _(cc=cc, nrb=nrb):
            fire_g(l, sa, (cc - 14) * _CH, nrb)
      return carry

    fire_group(0, sa, da, wa)
    wait_group(sa, da, wa, loff)
    fire_g(l, sa, 0, r0)
    fire_g(l, sa, _CH, r1)
    lax.fori_loop(0, nb, body, 0)
    wait_scatter()
    plsc.subcore_barrier()
    pltpu.sync_copy(acc.at[pl.ds(rs, _TR)],
                    xall.at[pl.ds(l * _XR + lo + rs, _TR)])
    pltpu.sync_copy(zer, acc.at[pl.ds(rs, _TR)])
    plsc.subcore_barrier()
    return lcarry

  lax.fori_loop(0, 3, layer, 0)

  pltpu.sync_copy(usr.at[pl.ds(c * 32 + s * 2, 2)], uidx)
  pltpu.sync_copy(itm.at[pl.ds(c * 32 + s * 2, 2)], iidx)
  for q in range(2):
    for k in range(4):
      iidx[q, pl.ds(k * 16, 16)] = iidx[q, pl.ds(k * 16, 16)] + _N_USERS
  perms = [(lanev + sh) & 15 for sh in (8, 4, 2, 1)]

  def _hsum(v):
    for p in perms:
      v = v + vperm(v, p)
    return v

  def gather_batch(idx_ref, q, l, rbuf):
    if l == 0:
      pltpu.async_copy(x0.at[idx_ref.at[q]], rbuf, gsem).wait()
    else:
      for k in range(4):
        dsc0[pl.ds(k * 16, 16)] = idx_ref[q, pl.ds(k * 16, 16)] + ((l - 1) * _XR)
      pltpu.async_copy(xall.at[dsc0], rbuf, gsem).wait()

  for q in range(2):
    for l in range(4):
      gather_batch(uidx, q, l, rbufs[min(l, 1)])
      if l >= 1:
        def usum(g, cy):
          for k in range(4):
            p = g * 4 + k
            for h in (0, 16, 32, 48):
              r0[p, pl.ds(h, 16)] = r0[p, pl.ds(h, 16)] + r1[p, pl.ds(h, 16)]
          return cy

        lax.fori_loop(0, 16, usum, 0)
    for l in range(4):
      gather_batch(iidx, q, l, r1)

      def dot(t, cy, q=q, first=(l == 0)):
        m = jnp.zeros((16,), jnp.float32)
        for h in (0, 16, 32, 48):
          m = m + r0[t, pl.ds(h, 16)] * r1[t, pl.ds(h, 16)]
        hs = _hsum(m)
        base = q * 64 + (t & ~15)
        av = gbuf[pl.ds(base, 16)]
        if first:
          gbuf[pl.ds(base, 16)] = jnp.where(lanev == (t & 15), hs, av)
        else:
          gbuf[pl.ds(base, 16)] = av + jnp.where(lanev == (t & 15), hs, 0.0)
        return cy

      lax.fori_loop(0, 64, dot, 0)

  def gscale(g, cy):
    gbuf[pl.ds(g * 16, 16)] = gbuf[pl.ds(g * 16, 16)] * 0.0625
    return cy

  lax.fori_loop(0, 8, gscale, 0)
  pltpu.sync_copy(gbuf, gout.at[c, 0, pl.ds(s * 128, 128)])


def _make_kernel():
  mesh = plsc.VectorSubcoreMesh(core_axis_name="c", subcore_axis_name="s")
  out_type = [
      jax.ShapeDtypeStruct((3 * _XR, _D), jnp.float32),
      jax.ShapeDtypeStruct((2, 1, 2048), jnp.float32),
  ]
  scratch = [
      pltpu.VMEM_SHARED((_HN + 2048, _D), jnp.float32),
      pltpu.VMEM((_CH, _D), jnp.float32),
      pltpu.VMEM((_CH, _D), jnp.float32),
      pltpu.VMEM((_CH, _D), jnp.float32),
      pltpu.VMEM((_CH, _D), jnp.float32),
      pltpu.VMEM((512,), jnp.int32),
      pltpu.VMEM((512,), jnp.int32),
      pltpu.VMEM((512,), jnp.float32),
      pltpu.VMEM((512,), jnp.int32),
      pltpu.VMEM((512,), jnp.int32),
      pltpu.VMEM((512,), jnp.float32),
      pltpu.VMEM((_CH,), jnp.int32),
      pltpu.VMEM((_CH,), jnp.int32),
      pltpu.VMEM((2, 64), jnp.int32),
      pltpu.VMEM((2, 64), jnp.int32),
      pltpu.VMEM((128,), jnp.float32),
      pltpu.SemaphoreType.DMA,
      pltpu.SemaphoreType.DMA,
      pltpu.SemaphoreType.DMA,
  ]
  return pl.kernel(_sc_body, out_type=out_type, mesh=mesh,
                   scratch_types=scratch,
                   compiler_params=pltpu.CompilerParams(
                       use_tc_tiling_on_sc=False))


_KERNEL = _make_kernel()


@jax.jit
def kernel(user_emb, item_emb, edge_index, edge_weight, users, items):
  x0 = jnp.concatenate(
      [user_emb, item_emb, jnp.zeros((_XR - _N_NODES, _D), jnp.float32)],
      axis=0)
  pad = _EPAD - _E
  srcp = jnp.concatenate([edge_index[0], jnp.zeros((pad,), jnp.int32)])
  dstp = jnp.concatenate([edge_index[1], jnp.zeros((pad,), jnp.int32)])
  wp = jnp.concatenate([edge_weight, jnp.zeros((pad,), jnp.float32)])
  usr = users.reshape(_B // 64, 64)
  itm = items.reshape(_B // 64, 64)
  zer = jnp.zeros((_TR, _D), jnp.float32)
  outs = _KERNEL(x0, srcp, dstp, wp, usr, itm, zer)
  gout = outs[1]
  return jnp.concatenate([gout[0, 0], gout[1, 0]])


# speedup vs baseline: 2.7185x; 2.7185x over previous
import jax
import jax.numpy as jnp
from jax import lax
from jax.experimental import pallas as pl
from jax.experimental.pallas import tpu as pltpu
from jax.experimental.pallas import tpu_sc as plsc

_N_USERS = 30000
_N_NODES = 50000
_E = 800000
_H = 32
_NSUB = 16
_EPAD = 819200
_ET = _EPAD // _NSUB
_CH = 128
_GCH = 8
_NB = 25
_RH = 51200
_TR = _RH // _NSUB
_B = 4096


def _sc_body(x0, srcp, dstp, wp, usr, itm, zer, x1, x2, x3, gpart,
             acc, r0, r1, r2, r3,
             sidxa, didxa, wbufa, sidxb, didxb, wbufb, gbuf,
             gsem, ssem, isem):
  c = lax.axis_index("c")
  s = lax.axis_index("s")
  coff = c * _RH
  rbufs = (r0, r1, r2, r3)
  rs = s * _TR
  irow0 = s * (_ET // _CH)

  pltpu.sync_copy(zer, acc.at[pl.ds(rs, _TR)])
  plsc.subcore_barrier()

  def fire_group_load(m, sb, db, wb):
    ir = irow0 + m * _GCH
    pltpu.async_copy(srcp.at[pl.ds(ir, _GCH)], sb, isem)
    pltpu.async_copy(dstp.at[pl.ds(ir, _GCH)], db, isem)
    pltpu.async_copy(wp.at[pl.ds(s * _ET + m * _GCH * _CH, _GCH * _CH)], wb, isem)

  def wait_group_load(sb, db, wb):
    pltpu.make_async_copy(srcp.at[pl.ds(0, _GCH)], sb, isem).wait()
    pltpu.make_async_copy(dstp.at[pl.ds(0, _GCH)], db, isem).wait()
    pltpu.make_async_copy(wp.at[pl.ds(0, _GCH * _CH)], wb, isem).wait()

  def offset_group(sb):
    def offr(r, cy):
      for k in range(8):
        sb[r, pl.ds(k * 16, 16)] = sb[r, pl.ds(k * 16, 16)] + coff
      return cy

    lax.fori_loop(0, _GCH, offr, 0)

  def load_group_sync(m, sb, db, wb):
    fire_group_load(m, sb, db, wb)
    wait_group_load(sb, db, wb)
    offset_group(sb)

  def fire_g(xin, sb, r, rbuf):
    pltpu.async_copy(xin.at[sb.at[r]], rbuf, gsem)

  def wait_g(xin, rbuf):
    pltpu.make_async_copy(xin.at[pl.ds(0, _CH)], rbuf, gsem).wait()

  def scale_chunk(rbuf, wb, woff):
    def scale(g, cy):
      wv = wb[pl.ds(woff + g * 16, 16)]
      for k in range(16):
        e = g * 16 + k
        w = wv[k]
        rbuf[e, pl.ds(0, 16)] = rbuf[e, pl.ds(0, 16)] * w
        rbuf[e, pl.ds(16, 16)] = rbuf[e, pl.ds(16, 16)] * w
      return cy

    lax.fori_loop(0, _CH // 16, scale, 0)

  def fire_scatter(rbuf, db, r):
    pltpu.async_copy(rbuf, acc.at[db.at[r]], ssem, add=True)

  def wait_scatter():
    pltpu.make_async_copy(r0, acc.at[pl.ds(0, _CH)], ssem).wait()

  for xin, xout in ((x0, x1), (x1, x2), (x2, x3)):

    def body(b, carry, xin=xin):
      for cc in range(16):
        rb = rbufs[cc % 4]
        grp_b = cc >= 8
        sb, db, wb = (sidxb, didxb, wbufb) if grp_b else (sidxa, didxa, wbufa)
        row = cc % 8
        if cc == 2:
          fire_group_load(2 * b + 1, sidxb, didxb, wbufb)
        wait_g(xin, rb)
        scale_chunk(rb, wb, row * _CH)
        if cc == 0:
          @pl.when(b > 0)
          def _():
            wait_scatter()
        else:
          wait_scatter()
        fire_scatter(rb, db, row)
        if cc == 5:
          wait_group_load(sidxb, didxb, wbufb)
          offset_group(sidxb)
        if cc == 10:
          @pl.when(b < _NB - 1)
          def _():
            fire_group_load(2 * b + 2, sidxa, didxa, wbufa)
        if cc == 13:
          @pl.when(b < _NB - 1)
          def _():
            wait_group_load(sidxa, didxa, wbufa)
            offset_group(sidxa)
        nrb = rbufs[(cc + 2) % 4]
        if cc < 6:
          fire_g(xin, sidxa, cc + 2, nrb)
        elif cc < 14:
          fire_g(xin, sidxb, cc - 6, nrb)
        else:
          @pl.when(b < _NB - 1)
          def _(cc=cc, nrb=nrb):
            fire_g(xin, sidxa, cc - 14, nrb)
      return carry

    load_group_sync(0, sidxa, didxa, wbufa)
    fire_g(xin, sidxa, 0, r0)
    fire_g(xin, sidxa, 1, r1)
    lax.fori_loop(0, _NB, body, 0)
    wait_scatter()
    plsc.subcore_barrier()
    pltpu.sync_copy(acc.at[pl.ds(rs, _TR)], xout.at[pl.ds(coff + rs, _TR)])
    pltpu.sync_copy(zer, acc.at[pl.ds(rs, _TR)])
    plsc.subcore_barrier()

  pltpu.sync_copy(usr.at[pl.ds(s * 2, 2)], sidxa.at[pl.ds(0, 2)])
  pltpu.sync_copy(itm.at[pl.ds(s * 2, 2)], didxa.at[pl.ds(0, 2)])
  for q in range(2):
    for k in range(8):
      sidxa[q, pl.ds(k * 16, 16)] = sidxa[q, pl.ds(k * 16, 16)] + coff
      didxa[q, pl.ds(k * 16, 16)] = didxa[q, pl.ds(k * 16, 16)] + (coff + _N_USERS)
  lane = lax.iota(jnp.int32, 16)
  perms = [(lane + sh) & 15 for sh in (8, 4, 2, 1)]
  gdims = lax.GatherDimensionNumbers(
      offset_dims=(), collapsed_slice_dims=(0,), start_index_map=(0,))

  def _hsum(v):
    for p in perms:
      v = v + lax.gather(v, p[:, None], gdims, (1,),
                         mode=lax.GatherScatterMode.PROMISE_IN_BOUNDS)
    return v

  xs = (x0, x1, x2, x3)
  for q in range(2):
    cps = [pltpu.async_copy(xs[l].at[sidxa.at[q]], rbufs[l], gsem)
           for l in range(4)]
    for cp in cps:
      cp.wait()

    def usum(g, cy):
      for k in range(8):
        p = g * 8 + k
        for h in (0, 16):
          r0[p, pl.ds(h, 16)] = (r0[p, pl.ds(h, 16)] + r1[p, pl.ds(h, 16)]
                                 + r2[p, pl.ds(h, 16)] + r3[p, pl.ds(h, 16)])
      return cy

    lax.fori_loop(0, 16, usum, 0)
    cps = [pltpu.async_copy(xs[l].at[didxa.at[q]], rbufs[l + 1], gsem)
           for l in range(3)]
    for cp in cps:
      cp.wait()

    def dot1(t, cy, q=q):
      ilo = r1[t, pl.ds(0, 16)] + r2[t, pl.ds(0, 16)] + r3[t, pl.ds(0, 16)]
      ihi = r1[t, pl.ds(16, 16)] + r2[t, pl.ds(16, 16)] + r3[t, pl.ds(16, 16)]
      hs = _hsum(r0[t, pl.ds(0, 16)] * ilo + r0[t, pl.ds(16, 16)] * ihi)
      base = q * 128 + (t & ~15)
      av = gbuf[pl.ds(base, 16)]
      gbuf[pl.ds(base, 16)] = jnp.where(lane == (t & 15), hs, av)
      return cy

    lax.fori_loop(0, 128, dot1, 0)
    pltpu.async_copy(xs[3].at[didxa.at[q]], r1, gsem).wait()

    def dot2(t, cy, q=q):
      hs = _hsum(r0[t, pl.ds(0, 16)] * r1[t, pl.ds(0, 16)]
                 + r0[t, pl.ds(16, 16)] * r1[t, pl.ds(16, 16)])
      base = q * 128 + (t & ~15)
      av = gbuf[pl.ds(base, 16)]
      gbuf[pl.ds(base, 16)] = av + jnp.where(lane == (t & 15), hs, 0.0)
      return cy

    lax.fori_loop(0, 128, dot2, 0)

  def gscale(g, cy):
    gbuf[pl.ds(g * 16, 16)] = gbuf[pl.ds(g * 16, 16)] * 0.0625
    return cy

  lax.fori_loop(0, 16, gscale, 0)
  pltpu.sync_copy(gbuf, gpart.at[c, 0, pl.ds(s * 256, 256)])


def _make_kernel():
  mesh = plsc.VectorSubcoreMesh(core_axis_name="c", subcore_axis_name="s")
  out_type = [
      jax.ShapeDtypeStruct((2 * _RH, _H), jnp.float32),
      jax.ShapeDtypeStruct((2 * _RH, _H), jnp.float32),
      jax.ShapeDtypeStruct((2 * _RH, _H), jnp.float32),
      jax.ShapeDtypeStruct((2, 1, _B), jnp.float32),
  ]
  scratch = [
      pltpu.VMEM_SHARED((_RH, _H), jnp.float32),
      pltpu.VMEM((_CH, _H), jnp.float32),
      pltpu.VMEM((_CH, _H), jnp.float32),
      pltpu.VMEM((_CH, _H), jnp.float32),
      pltpu.VMEM((_CH, _H), jnp.float32),
      pltpu.VMEM((_GCH, 128), jnp.int32),
      pltpu.VMEM((_GCH, 128), jnp.int32),
      pltpu.VMEM((_GCH * _CH,), jnp.float32),
      pltpu.VMEM((_GCH, 128), jnp.int32),
      pltpu.VMEM((_GCH, 128), jnp.int32),
      pltpu.VMEM((_GCH * _CH,), jnp.float32),
      pltpu.VMEM((256,), jnp.float32),
      pltpu.SemaphoreType.DMA,
      pltpu.SemaphoreType.DMA,
      pltpu.SemaphoreType.DMA,
  ]
  return pl.kernel(_sc_body, out_type=out_type, mesh=mesh,
                   scratch_types=scratch,
                   compiler_params=pltpu.CompilerParams(
                       use_tc_tiling_on_sc=False))


_KERNEL = _make_kernel()


@jax.jit
def kernel(user_emb, item_emb, edge_index, edge_weight, users, items):
  all_emb = jnp.concatenate([user_emb, item_emb], axis=0)
  zpad = jnp.zeros((_RH - _N_NODES, _H), jnp.float32)
  x0 = jnp.concatenate(
      [all_emb[:, :_H], zpad, all_emb[:, _H:], zpad], axis=0)
  pad = _EPAD - _E
  srcp = jnp.concatenate(
      [edge_index[0], jnp.zeros((pad,), jnp.int32)]).reshape(_EPAD // 128, 128)
  dstp = jnp.concatenate(
      [edge_index[1], jnp.zeros((pad,), jnp.int32)]).reshape(_EPAD // 128, 128)
  wp = jnp.concatenate([edge_weight, jnp.zeros((pad,), jnp.float32)])
  usr = users.reshape(_B // 128, 128)
  itm = items.reshape(_B // 128, 128)
  zer = jnp.zeros((_TR, _H), jnp.float32)
  _, _, _, gpart = _KERNEL(x0, srcp, dstp, wp, usr, itm, zer)
  return gpart[0, 0] + gpart[1, 0]
